# baseline (speedup 1.0000x reference)
import functools

import jax
import jax.numpy as jnp
from jax import lax
from jax.experimental import pallas as pl
from jax.experimental.pallas import tpu as pltpu
from jax.experimental.pallas import tpu_sc as plsc

_H = 20
_WD = 20
_F = 32
_K = 8
_N = _H * _WD
_NC = 2
_NS = 16
_CELLS_PER_W = _WD
_ROWS_PER_W = _CELLS_PER_W * _K
_GATHER_SPLIT = 80
_LANES = 16


def _mm_body(x_ref, w_ref, xw_ref):
    w = w_ref[...]
    for i in range(_H):
        xw_ref[pl.ds(i * _WD, _WD), pl.ds(0, _F)] = jnp.dot(
            x_ref[i], w, preferred_element_type=jnp.float32)


def _project(x, w):
    return pl.pallas_call(
        _mm_body,
        out_shape=jax.ShapeDtypeStruct((_N, 128), jnp.float32),
    )(x, w)


def _sc_body(xw_hbm, nn_hbm, out_hbm, nnv, idx_v, rows_v, out_v, sem):
    wid = lax.axis_index("s") * _NC + lax.axis_index("c")

    @pl.when(wid < _H)
    def _():
        pltpu.sync_copy(
            nn_hbm.at[pl.ds(wid * _ROWS_PER_W * 3, _ROWS_PER_W * 3)], nnv)
        lane = lax.iota(jnp.int32, _LANES)
        b_vec = lane & (_K - 1)
        half_cell = lane >> 3
        kvalid = b_vec < _K - 1
        sim_regs = []
        for v in range(_ROWS_PER_W // _LANES):
            a_vec = 2 * v + half_cell
            pos = lane * 3 + (v * 3 * _LANES + 1)
            idf = plsc.load_gather(nnv, [pos])
            sif = plsc.load_gather(nnv, [pos + 1])
            valid = kvalid & (a_vec < _WD - 1) & (wid < _H - 1)
            sim_regs.append(jnp.where(valid, sif, jnp.float32(0.0)))
            idx_v[pl.ds(v * _LANES, _LANES)] = idf.astype(jnp.int32)
        cp0 = pltpu.async_copy(
            xw_hbm.at[idx_v.at[pl.ds(0, _GATHER_SPLIT)]],
            rows_v.at[pl.ds(0, _GATHER_SPLIT)], sem)
        cp1 = pltpu.async_copy(
            xw_hbm.at[idx_v.at[pl.ds(_GATHER_SPLIT, _GATHER_SPLIT)]],
            rows_v.at[pl.ds(_GATHER_SPLIT, _GATHER_SPLIT)], sem)
        cp0.wait()
        cp1.wait()
        for v in range(_ROWS_PER_W // _LANES):
            sv = sim_regs[v]
            for half, c in ((0, 2 * v), (_K, 2 * v + 1)):
                r0 = c * _K
                s = sv[half]
                acc_lo = s * rows_v[r0, pl.ds(0, _LANES)]
                acc_hi = s * rows_v[r0, pl.ds(_LANES, _LANES)]
                for k in range(1, _K):
                    r = r0 + k
                    s = sv[half + k]
                    acc_lo = acc_lo + s * rows_v[r, pl.ds(0, _LANES)]
                    acc_hi = acc_hi + s * rows_v[r, pl.ds(_LANES, _LANES)]
                out_v[c, pl.ds(0, _LANES)] = acc_lo
                out_v[c, pl.ds(_LANES, _LANES)] = acc_hi
        pltpu.sync_copy(out_v, out_hbm.at[wid])


_sc_gather = functools.partial(
    pl.kernel,
    out_type=jax.ShapeDtypeStruct((_H, _WD, _F), jnp.float32),
    mesh=plsc.VectorSubcoreMesh(core_axis_name="c", subcore_axis_name="s",
                                num_cores=_NC, num_subcores=_NS),
    scratch_types=[
        pltpu.VMEM((_ROWS_PER_W * 3,), jnp.float32),
        pltpu.VMEM((_ROWS_PER_W,), jnp.int32),
        pltpu.VMEM((_ROWS_PER_W, 128), jnp.float32),
        pltpu.VMEM((_CELLS_PER_W, _F), jnp.float32),
        pltpu.SemaphoreType.DMA,
    ],
    compiler_params=pltpu.CompilerParams(use_tc_tiling_on_sc=False,
                                         needs_layout_passes=False),
)(_sc_body)


def kernel(x, W, nearest_neighbors):
    xw = _project(x, W)
    return _sc_gather(xw, nearest_neighbors.reshape(_N * _K * 3))


# speedup vs baseline: 1.0632x; 1.0632x over previous
import functools

import jax
import jax.numpy as jnp
from jax import lax
from jax.experimental import pallas as pl
from jax.experimental.pallas import tpu as pltpu
from jax.experimental.pallas import tpu_sc as plsc

_H = 20
_WD = 20
_F = 32
_K = 8
_N = _H * _WD
_NC = 2
_NS = 16
_CELLS_PER_W = _WD
_ROWS_PER_W = _CELLS_PER_W * _K
_GATHER_SPLIT = 80
_LANES = 16


def _mm_body(x_ref, w_ref, xw_ref):
    w = w_ref[...]
    for i in range(_H):
        xw_ref[pl.ds(i * _WD, _WD), pl.ds(0, _F)] = jnp.dot(
            x_ref[i], w, preferred_element_type=jnp.float32)


def _project(x, w):
    return pl.pallas_call(
        _mm_body,
        out_shape=jax.ShapeDtypeStruct((_N, 128), jnp.float32),
    )(x, w)


def _sc_body(xw_hbm, nn_hbm, out_hbm, nnv, idx_v, rows_v, out_v, sem):
    wid = lax.axis_index("s") * _NC + lax.axis_index("c")

    @pl.when(wid < _H)
    def _():
        pltpu.sync_copy(
            nn_hbm.at[pl.ds(wid * _ROWS_PER_W * 3, _ROWS_PER_W * 3)], nnv)
        lane = lax.iota(jnp.int32, _LANES)
        b_vec = lane & (_K - 1)
        half_cell = lane >> 3
        kvalid = b_vec < _K - 1
        sim_regs = []
        for v in range(_ROWS_PER_W // _LANES):
            a_vec = 2 * v + half_cell
            pos = lane * 3 + (v * 3 * _LANES + 1)
            idf = plsc.load_gather(nnv, [pos])
            sif = plsc.load_gather(nnv, [pos + 1])
            valid = kvalid & (a_vec < _WD - 1) & (wid < _H - 1)
            sim_regs.append(jnp.where(valid, sif, jnp.float32(0.0)))
            idx_v[pl.ds(v * _LANES, _LANES)] = idf.astype(jnp.int32)
        cp0 = pltpu.async_copy(
            xw_hbm.at[idx_v.at[pl.ds(0, _GATHER_SPLIT)]],
            rows_v.at[pl.ds(0, _GATHER_SPLIT)], sem)
        cp1 = pltpu.async_copy(
            xw_hbm.at[idx_v.at[pl.ds(_GATHER_SPLIT, _GATHER_SPLIT)]],
            rows_v.at[pl.ds(_GATHER_SPLIT, _GATHER_SPLIT)], sem)
        cp0.wait()
        cp1.wait()
        for v in range(_ROWS_PER_W // _LANES):
            sv = sim_regs[v]
            for half, c in ((0, 2 * v), (_K, 2 * v + 1)):
                r0 = c * _K
                s = sv[half]
                acc_lo = s * rows_v[r0, pl.ds(0, _LANES)]
                acc_hi = s * rows_v[r0, pl.ds(_LANES, _LANES)]
                for k in range(1, _K):
                    r = r0 + k
                    s = sv[half + k]
                    acc_lo = acc_lo + s * rows_v[r, pl.ds(0, _LANES)]
                    acc_hi = acc_hi + s * rows_v[r, pl.ds(_LANES, _LANES)]
                out_v[c, pl.ds(0, _LANES)] = acc_lo
                out_v[c, pl.ds(_LANES, _LANES)] = acc_hi
        pltpu.sync_copy(out_v, out_hbm.at[wid])


_sc_gather = functools.partial(
    pl.kernel,
    out_type=jax.ShapeDtypeStruct((_H, _WD, _F), jnp.float32),
    mesh=plsc.VectorSubcoreMesh(core_axis_name="c", subcore_axis_name="s",
                                num_cores=_NC, num_subcores=_NS),
    scratch_types=[
        pltpu.VMEM((_ROWS_PER_W * 3,), jnp.float32),
        pltpu.VMEM((_ROWS_PER_W,), jnp.int32),
        pltpu.VMEM((_ROWS_PER_W, 128), jnp.float32),
        pltpu.VMEM((_CELLS_PER_W, _F), jnp.float32),
        pltpu.SemaphoreType.DMA,
    ],
    compiler_params=pltpu.CompilerParams(use_tc_tiling_on_sc=True,
                                         needs_layout_passes=False),
)(_sc_body)


def kernel(x, W, nearest_neighbors):
    xw = _project(x, W)
    return _sc_gather(xw, nearest_neighbors.reshape(_N * _K * 3))
